# baseline (speedup 1.0000x reference)
import functools

import jax
import jax.numpy as jnp
from jax import lax
from jax.experimental import pallas as pl
from jax.experimental.pallas import tpu as pltpu
from jax.experimental.pallas import tpu_sc as plsc

NC = 2
NS = 16
NW = NC * NS
DIM = 32
GC = 256
G = GC // 128


def _ceil_div(a, b):
  return -(-a // b)


def _proj_body(cfg, tu, th, tut, tht, tl, tr, tp, w_hbm_ref,
               pu, pl_, ph, pr, pp,
               slab0, slab1, slab2, slab3, stage, w_v, sem0, sem1, sem2, sem3):
  (nbu, nbh) = cfg
  slabs = [slab0, slab1, slab2, slab3]
  sems = [sem0, sem1, sem2, sem3]

  NT_U, NT_H = 128, 16
  T_EXTRA = NT_U + NT_H
  NT = 148
  SOFF_H, SOFF_E, SOFF_T = NT_U * GC, (NT_U + NT_H) * GC, (NT_U + NT_H + 1) * GC

  wid = lax.axis_index("s") * NC + lax.axis_index("c")
  pltpu.sync_copy(w_hbm_ref, w_v)

  ucols, hcols = nbu * 128, nbh * 128
  stride_u = _ceil_div(_ceil_div(ucols, NW), 128) * 128
  stride_h = _ceil_div(_ceil_div(hcols, NW), 128) * 128
  start_u = pl.multiple_of(jnp.minimum(wid * stride_u, ucols - NT_U * GC), 128)
  start_h = pl.multiple_of(jnp.minimum(wid * stride_h, hcols - NT_H * GC), 128)

  e_tid = jnp.where(wid == 0, 0,
          jnp.where(wid == 1, 2,
          jnp.where(wid < 6, 1, jnp.where(wid < 10, 3,
          jnp.where(wid < 14, 4, 0)))))
  e_branch = jnp.where(wid == 0, 2, jnp.where(wid == 1, 3,
             jnp.where(wid < 6, 4, jnp.where(wid < 10, 5,
             jnp.where(wid < 14, 6, 0)))))
  quarter = pl.multiple_of(jnp.where((wid >= 2) & (wid < 14),
                                     ((wid - 2) % 4) * GC, 0), 16)

  def fire_task(k, p):
    cu = pl.multiple_of(jnp.minimum(start_u + k * GC, ucols - GC), 128)
    ch = pl.multiple_of(jnp.minimum(start_h + (k - NT_U) * GC, hcols - GC), 128)
    ch = pl.multiple_of(jnp.maximum(ch, 0), 128)

    def mk(tab, c):
      def go():
        pltpu.async_copy(tab.at[pl.ds(0, 32), pl.ds(c, GC)],
                         slabs[p], sems[p])
      return go

    branch = jnp.where(k < NT_U, 0, jnp.where(k < T_EXTRA, 1, e_branch))
    lax.switch(branch, [mk(tu, cu), mk(th, ch),
                        mk(tut, pl.multiple_of(jnp.int32(0), 128)),
                        mk(tht, pl.multiple_of(jnp.int32(0), 128)),
                        mk(tl, quarter), mk(tr, quarter), mk(tp, quarter)])

  def drain(p):
    pltpu.make_async_copy(tu.at[pl.ds(0, 32), pl.ds(0, GC)],
                          slabs[p].at[pl.ds(0, 32), pl.ds(0, GC)],
                          sems[p]).wait()

  def compute_task(k, p):
    t_id = jnp.where(k < NT_U, 0, jnp.where(k < T_EXTRA, 2, e_tid))
    toff = pl.multiple_of(t_id * DIM, 16)
    wv0 = w_v[pl.ds(toff, 16)]
    wv1 = w_v[pl.ds(toff + 16, 16)]
    ws = [wv0[d] for d in range(16)] + [wv1[d] for d in range(16)]
    soff = jnp.where(k < NT_U, k * GC,
           jnp.where(k < T_EXTRA, SOFF_H + (k - NT_U) * GC,
           jnp.where(k == T_EXTRA, SOFF_E, SOFF_T)))
    soff = pl.multiple_of(soff, 16)
    acc = [slabs[p][0, pl.ds(v * 16, 16)] * ws[0] for v in range(GC // 16)]
    for d in range(1, DIM):
      for v in range(GC // 16):
        acc[v] = acc[v] + slabs[p][d, pl.ds(v * 16, 16)] * ws[d]
    for v in range(GC // 16):
      stage[pl.ds(soff + v * 16, 16)] = acc[v]

  for p in range(4):
    fire_task(jnp.int32(p), p)

  def body(it, carry):
    for p in range(4):
      k = 4 * it + p
      compute_task(k, p)
    return carry

  lax.fori_loop(0, NT // 4, body, 0)
  for p in range(4):
    drain(p)

  pltpu.sync_copy(stage.at[pl.ds(0, NT_U * GC)],
                  pu.at[pl.ds(start_u, NT_U * GC)])
  pltpu.sync_copy(stage.at[pl.ds(SOFF_H, NT_H * GC)],
                  ph.at[pl.ds(start_h, NT_H * GC)])

  @pl.when(wid == 0)
  def _():
    pltpu.sync_copy(stage.at[pl.ds(SOFF_E, GC)], pu.at[pl.ds(ucols, GC)])

  @pl.when(wid == 1)
  def _():
    pltpu.sync_copy(stage.at[pl.ds(SOFF_E, GC)], ph.at[pl.ds(hcols, GC)])

  @pl.when((wid >= 2) & (wid < 14))
  def _():
    dst = jnp.where(wid < 6, 0, jnp.where(wid < 10, 1, 2))

    def mk(proj):
      def go():
        pltpu.sync_copy(stage.at[pl.ds(SOFF_E, GC)], proj.at[pl.ds(quarter, GC)])
      return go

    lax.switch(dst, [mk(pl_), mk(pr), mk(pp)])


def _gather_body(bpw, p0, p1, p2, p3, p4, i0, i1, i2, i3, i4, b_hbm, out_hbm,
                 v0, v1, v2, v3, v4, r0, r1, r2, r3, r4, b_v, out_v, sem):
  projs = [p0, p1, p2, p3, p4]
  idx_hbms = [i0, i1, i2, i3, i4]
  idx_vs = [v0, v1, v2, v3, v4]
  rows = [r0, r1, r2, r3, r4]

  wid = lax.axis_index("s") * NC + lax.axis_index("c")
  base = pl.multiple_of(wid * bpw, bpw)
  pltpu.sync_copy(b_hbm, b_v)
  for t in range(5):
    pltpu.sync_copy(idx_hbms[t].at[pl.ds(base, bpw)], idx_vs[t])
  copies = []
  for t in range(5):
    for j in range(bpw // 128):
      copies.append(pltpu.async_copy(
          projs[t].at[idx_vs[t].at[pl.ds(j * 128, 128)]],
          rows[t].at[pl.ds(j * 128, 128)], sem))
  for c in copies:
    c.wait()
  bvec = b_v[...]
  for c in range(bpw // 16):
    o = rows[0][pl.ds(c * 16, 16)] + bvec
    for t in range(1, 5):
      o = o + rows[t][pl.ds(c * 16, 16)]
    out_v[pl.ds(c * 16, 16)] = o
  pltpu.sync_copy(out_v, out_hbm.at[pl.ds(base, bpw)])


def kernel(user, location, hotel, hotelrating, price,
           user_table, location_table, hotel_table, rating_table, price_table,
           W, b):
  batch = user.shape[0]
  vu, vh = user_table.shape[0], hotel_table.shape[0]
  vsml = location_table.shape[0]
  nbu, nbh = vu // 128, vh // 128
  bpw = batch // NW

  ut_tail = jnp.pad(user_table[nbu * 128:], ((0, 256 - (vu - nbu * 128)), (0, 0))).T
  ht_tail = jnp.pad(hotel_table[nbh * 128:], ((0, 256 - (vh - nbh * 128)), (0, 0))).T
  lp = jnp.pad(location_table, ((0, 1024 - vsml), (0, 0))).T
  rp = jnp.pad(rating_table, ((0, 1024 - vsml), (0, 0))).T
  pp = jnp.pad(price_table, ((0, 1024 - vsml), (0, 0))).T

  pu_n, ph_n = nbu * 128 + 256, nbh * 128 + 256

  mesh = plsc.VectorSubcoreMesh(core_axis_name="c", subcore_axis_name="s",
                                num_cores=NC, num_subcores=NS)

  proj_fn = pl.kernel(
      functools.partial(_proj_body, (nbu, nbh)),
      out_type=tuple(jax.ShapeDtypeStruct((n,), jnp.float32)
                     for n in (pu_n, 1024, ph_n, 1024, 1024)),
      mesh=mesh,
      compiler_params=pltpu.CompilerParams(needs_layout_passes=False,
                                           use_tc_tiling_on_sc=True),
      scratch_types=(
          [pltpu.VMEM((32, GC), jnp.float32) for _ in range(4)]
          + [pltpu.VMEM((146 * GC,), jnp.float32),
             pltpu.VMEM((5 * DIM,), jnp.float32)]
          + [pltpu.SemaphoreType.DMA for _ in range(4)]),
  )
  projs = proj_fn(user_table.T, hotel_table.T, ut_tail, ht_tail,
                  lp, rp, pp, W.reshape(5 * DIM))

  gather_fn = pl.kernel(
      functools.partial(_gather_body, bpw),
      out_type=jax.ShapeDtypeStruct((batch,), jnp.float32),
      mesh=mesh,
      compiler_params=pltpu.CompilerParams(needs_layout_passes=False),
      scratch_types=(
          [pltpu.VMEM((bpw,), jnp.int32) for _ in range(5)]
          + [pltpu.VMEM((bpw,), jnp.float32) for _ in range(5)]
          + [pltpu.VMEM((16,), jnp.float32),
             pltpu.VMEM((bpw,), jnp.float32),
             pltpu.SemaphoreType.DMA]),
  )
  out = gather_fn(*projs,
                  user.astype(jnp.int32), location.astype(jnp.int32),
                  hotel.astype(jnp.int32), hotelrating.astype(jnp.int32),
                  price.astype(jnp.int32),
                  jnp.broadcast_to(b.astype(jnp.float32), (16,)))
  return out.reshape(batch, 1)


# speedup vs baseline: 2.6698x; 2.6698x over previous
import functools

import jax
import jax.numpy as jnp
from jax import lax
from jax.experimental import pallas as pl
from jax.experimental.pallas import tpu as pltpu
from jax.experimental.pallas import tpu_sc as plsc

NC = 2
NS = 16
NW = NC * NS
DIM = 32
GC = 128
G = GC // 128


def _ceil_div(a, b):
  return -(-a // b)


def _proj_body(cfg, tu, th, tut, tht, tl, tr, tp, w_hbm_ref,
               pu, pl_, ph, pr, pp,
               slab0, slab1, slab2, slab3, stage, w_v, sem0, sem1, sem2, sem3):
  (nbu, nbh) = cfg
  slabs = [slab0, slab1, slab2, slab3]
  sems = [sem0, sem1, sem2, sem3]

  NT_U, NT_H = 128, 16
  T_EXTRA = NT_U + NT_H
  NT = 148
  SOFF_H, SOFF_E, SOFF_T = NT_U * GC, (NT_U + NT_H) * GC, (NT_U + NT_H + 1) * GC

  wid = lax.axis_index("s") * NC + lax.axis_index("c")
  pltpu.sync_copy(w_hbm_ref, w_v)

  ucols, hcols = nbu * 128, nbh * 128
  stride_u = _ceil_div(_ceil_div(ucols, NW), 128) * 128
  stride_h = _ceil_div(_ceil_div(hcols, NW), 128) * 128
  start_u = pl.multiple_of(jnp.minimum(wid * stride_u, ucols - NT_U * GC), 128)
  start_h = pl.multiple_of(jnp.minimum(wid * stride_h, hcols - NT_H * GC), 128)

  e_tid = jnp.where(wid == 0, 0,
          jnp.where(wid == 1, 2,
          jnp.where(wid < 10, 1, jnp.where(wid < 18, 3,
          jnp.where(wid < 26, 4, 0)))))
  e_branch = jnp.where(wid == 0, 2, jnp.where(wid == 1, 3,
             jnp.where(wid < 10, 4, jnp.where(wid < 18, 5,
             jnp.where(wid < 26, 6, 0)))))
  quarter = pl.multiple_of(jnp.where((wid >= 2) & (wid < 26),
                                     ((wid - 2) % 8) * GC, 0), 16)

  def fire_task(k, p):
    cu = pl.multiple_of(jnp.minimum(start_u + k * GC, ucols - GC), 128)
    ch = pl.multiple_of(jnp.minimum(start_h + (k - NT_U) * GC, hcols - GC), 128)
    ch = pl.multiple_of(jnp.maximum(ch, 0), 128)

    def mk(tab, c):
      def go():
        pltpu.async_copy(tab.at[pl.ds(0, 32), pl.ds(c, GC)],
                         slabs[p], sems[p])
      return go

    branch = jnp.where(k < NT_U, 0, jnp.where(k < T_EXTRA, 1, e_branch))
    lax.switch(branch, [mk(tu, cu), mk(th, ch),
                        mk(tut, pl.multiple_of(jnp.int32(0), 128)),
                        mk(tht, pl.multiple_of(jnp.int32(0), 128)),
                        mk(tl, quarter), mk(tr, quarter), mk(tp, quarter)])

  def drain(p):
    pltpu.make_async_copy(tu.at[pl.ds(0, 32), pl.ds(0, GC)],
                          slabs[p].at[pl.ds(0, 32), pl.ds(0, GC)],
                          sems[p]).wait()

  def compute_task(k, p):
    t_id = jnp.where(k < NT_U, 0, jnp.where(k < T_EXTRA, 2, e_tid))
    toff = pl.multiple_of(t_id * DIM, 16)
    wv0 = w_v[pl.ds(toff, 16)]
    wv1 = w_v[pl.ds(toff + 16, 16)]
    ws = [wv0[d] for d in range(16)] + [wv1[d] for d in range(16)]
    soff = jnp.where(k < NT_U, k * GC,
           jnp.where(k < T_EXTRA, SOFF_H + (k - NT_U) * GC,
           jnp.where(k == T_EXTRA, SOFF_E, SOFF_T)))
    soff = pl.multiple_of(soff, 16)
    acc = [slabs[p][0, pl.ds(v * 16, 16)] * ws[0] for v in range(GC // 16)]
    for d in range(1, DIM):
      for v in range(GC // 16):
        acc[v] = acc[v] + slabs[p][d, pl.ds(v * 16, 16)] * ws[d]
    for v in range(GC // 16):
      stage[pl.ds(soff + v * 16, 16)] = acc[v]

  for p in range(4):
    fire_task(jnp.int32(p), p)

  def body(it, carry):
    for p in range(4):
      k = 4 * it + p
      drain(p)
      compute_task(k, p)
      fire_task(jnp.minimum(k + 4, NT - 1), p)
    return carry

  lax.fori_loop(0, NT // 4, body, 0)
  for p in range(4):
    drain(p)

  pltpu.sync_copy(stage.at[pl.ds(0, NT_U * GC)],
                  pu.at[pl.ds(start_u, NT_U * GC)])
  pltpu.sync_copy(stage.at[pl.ds(SOFF_H, NT_H * GC)],
                  ph.at[pl.ds(start_h, NT_H * GC)])

  @pl.when(wid == 0)
  def _():
    pltpu.sync_copy(stage.at[pl.ds(SOFF_E, GC)], pu.at[pl.ds(ucols, GC)])

  @pl.when(wid == 1)
  def _():
    pltpu.sync_copy(stage.at[pl.ds(SOFF_E, GC)], ph.at[pl.ds(hcols, GC)])

  @pl.when((wid >= 2) & (wid < 26))
  def _():
    dst = jnp.where(wid < 10, 0, jnp.where(wid < 18, 1, 2))

    def mk(proj):
      def go():
        pltpu.sync_copy(stage.at[pl.ds(SOFF_E, GC)], proj.at[pl.ds(quarter, GC)])
      return go

    lax.switch(dst, [mk(pl_), mk(pr), mk(pp)])


def _gather_body(bpw, p0, p1, p2, p3, p4, i0, i1, i2, i3, i4, b_hbm, out_hbm,
                 v0, v1, v2, v3, v4, r0, r1, r2, r3, r4, b_v, out_v, sem):
  projs = [p0, p1, p2, p3, p4]
  idx_hbms = [i0, i1, i2, i3, i4]
  idx_vs = [v0, v1, v2, v3, v4]
  rows = [r0, r1, r2, r3, r4]

  wid = lax.axis_index("s") * NC + lax.axis_index("c")
  base = pl.multiple_of(wid * bpw, bpw)
  pltpu.sync_copy(b_hbm, b_v)
  for t in range(5):
    pltpu.sync_copy(idx_hbms[t].at[pl.ds(base, bpw)], idx_vs[t])
  copies = []
  for t in range(5):
    for j in range(bpw // 128):
      copies.append(pltpu.async_copy(
          projs[t].at[idx_vs[t].at[pl.ds(j * 128, 128)]],
          rows[t].at[pl.ds(j * 128, 128)], sem))
  for c in copies:
    c.wait()
  bvec = b_v[...]
  for c in range(bpw // 16):
    o = rows[0][pl.ds(c * 16, 16)] + bvec
    for t in range(1, 5):
      o = o + rows[t][pl.ds(c * 16, 16)]
    out_v[pl.ds(c * 16, 16)] = o
  pltpu.sync_copy(out_v, out_hbm.at[pl.ds(base, bpw)])


def kernel(user, location, hotel, hotelrating, price,
           user_table, location_table, hotel_table, rating_table, price_table,
           W, b):
  batch = user.shape[0]
  vu, vh = user_table.shape[0], hotel_table.shape[0]
  vsml = location_table.shape[0]
  nbu, nbh = vu // 128, vh // 128
  bpw = batch // NW

  ut_tail = jnp.pad(user_table[nbu * 128:], ((0, 128 - (vu - nbu * 128)), (0, 0))).T
  ht_tail = jnp.pad(hotel_table[nbh * 128:], ((0, 128 - (vh - nbh * 128)), (0, 0))).T
  lp = jnp.pad(location_table, ((0, 1024 - vsml), (0, 0))).T
  rp = jnp.pad(rating_table, ((0, 1024 - vsml), (0, 0))).T
  pp = jnp.pad(price_table, ((0, 1024 - vsml), (0, 0))).T

  pu_n, ph_n = nbu * 128 + 128, nbh * 128 + 128

  mesh = plsc.VectorSubcoreMesh(core_axis_name="c", subcore_axis_name="s",
                                num_cores=NC, num_subcores=NS)

  proj_fn = pl.kernel(
      functools.partial(_proj_body, (nbu, nbh)),
      out_type=tuple(jax.ShapeDtypeStruct((n,), jnp.float32)
                     for n in (pu_n, 1024, ph_n, 1024, 1024)),
      mesh=mesh,
      compiler_params=pltpu.CompilerParams(needs_layout_passes=False,
                                           use_tc_tiling_on_sc=True),
      scratch_types=(
          [pltpu.VMEM((32, GC), jnp.float32) for _ in range(4)]
          + [pltpu.VMEM((292 * GC,), jnp.float32),
             pltpu.VMEM((5 * DIM,), jnp.float32)]
          + [pltpu.SemaphoreType.DMA for _ in range(4)]),
  )
  projs = proj_fn(user_table.T, hotel_table.T, ut_tail, ht_tail,
                  lp, rp, pp, W.reshape(5 * DIM))

  gather_fn = pl.kernel(
      functools.partial(_gather_body, bpw),
      out_type=jax.ShapeDtypeStruct((batch,), jnp.float32),
      mesh=mesh,
      compiler_params=pltpu.CompilerParams(needs_layout_passes=False),
      scratch_types=(
          [pltpu.VMEM((bpw,), jnp.int32) for _ in range(5)]
          + [pltpu.VMEM((bpw,), jnp.float32) for _ in range(5)]
          + [pltpu.VMEM((16,), jnp.float32),
             pltpu.VMEM((bpw,), jnp.float32),
             pltpu.SemaphoreType.DMA]),
  )
  out = gather_fn(*projs,
                  user.astype(jnp.int32), location.astype(jnp.int32),
                  hotel.astype(jnp.int32), hotelrating.astype(jnp.int32),
                  price.astype(jnp.int32),
                  jnp.broadcast_to(b.astype(jnp.float32), (16,)))
  return out.reshape(batch, 1)
